# baseline (speedup 1.0000x reference)
import functools

import jax
import jax.numpy as jnp
from jax import lax
from jax.experimental import pallas as pl
from jax.experimental.pallas import tpu as pltpu
from jax.experimental.pallas import tpu_sc as plsc

N = 10000
E = 320000
D = 128
W = 144
NPAD = 10240
NC = 2
NS = 16
NW = NC * NS
IDXW = 50
CPW = (E // IDXW) // NW
PGC = 40
NPG = CPW // PGC
NB = 4
RPT = NPAD // NS


def _sc_agg_body(xa_hbm, src_hbm, dst_hbm, zeros_hbm, out_hbm,
                 idx_s, idx_d, rows, shared, sem):
    cid = lax.axis_index("c")
    sid = lax.axis_index("s")
    wid = cid * NS + sid

    gsems = sem[:NB]
    ssems = sem[NB:2 * NB]
    isems = sem[2 * NB:]

    pltpu.sync_copy(zeros_hbm, shared.at[pl.ds(sid * RPT, RPT)])
    pltpu.sync_copy(src_hbm.at[wid, pl.ds(0, PGC)], idx_s.at[0])
    pltpu.sync_copy(dst_hbm.at[wid, pl.ds(0, PGC)], idx_d.at[0])
    plsc.subcore_barrier()

    def gather(pb, k, buf):
        pltpu.async_copy(xa_hbm.at[idx_s.at[pb, k]], rows.at[buf],
                         gsems[buf])

    def wait_gather(pb, k, buf):
        pltpu.make_async_copy(xa_hbm.at[idx_s.at[pb, k]], rows.at[buf],
                              gsems[buf]).wait()

    def scatter(pb, k, buf):
        pltpu.async_copy(rows.at[buf], shared.at[idx_d.at[pb, k]],
                         ssems[buf], add=True)

    def wait_scatter(pb, k, buf):
        pltpu.make_async_copy(rows.at[buf], shared.at[idx_d.at[pb, k]],
                              ssems[buf]).wait()

    for p in range(NPG):
        pb = p % 2
        if p > 0:
            pltpu.make_async_copy(src_hbm.at[wid, pl.ds(p * PGC, PGC)],
                                  idx_s.at[pb], isems[pb]).wait()
            pltpu.make_async_copy(dst_hbm.at[wid, pl.ds(p * PGC, PGC)],
                                  idx_d.at[pb], isems[pb]).wait()
        if p < NPG - 1:
            nb = (p + 1) % 2
            pltpu.async_copy(src_hbm.at[wid, pl.ds((p + 1) * PGC, PGC)],
                             idx_s.at[nb], isems[nb])
            pltpu.async_copy(dst_hbm.at[wid, pl.ds((p + 1) * PGC, PGC)],
                             idx_d.at[nb], isems[nb])

        gather(pb, 0, 0)
        gather(pb, 1, 1)
        for c in (0, 1):
            wait_gather(pb, c, c)
            scatter(pb, c, c)
            gather(pb, c + 2, c + 2)

        def step(k, carry, pb=pb):
            for m in range(NB):
                c = 2 + NB * k + m
                b = (2 + m) % NB
                wait_gather(pb, c, b)
                scatter(pb, c, b)
                wait_scatter(pb, c - 2, m % NB)
                gather(pb, c + 2, m % NB)
            return carry

        lax.fori_loop(0, (PGC - NB) // NB, step, 0)
        for c in (PGC - 2, PGC - 1):
            wait_gather(pb, c, c % NB)
            scatter(pb, c, c % NB)
        for c in range(PGC - NB, PGC):
            wait_scatter(pb, c, c % NB)
    plsc.subcore_barrier()

    pltpu.sync_copy(shared.at[pl.ds(sid * RPT, RPT)],
                    out_hbm.at[pl.ds(cid * NPAD + sid * RPT, RPT)])


def _make_sc_agg():
    mesh = plsc.VectorSubcoreMesh(core_axis_name="c", subcore_axis_name="s")
    return pl.kernel(
        _sc_agg_body,
        out_type=jax.ShapeDtypeStruct((2 * NPAD, W), jnp.float32),
        mesh=mesh,
        scratch_types=[
            pltpu.VMEM((2, PGC, IDXW), jnp.int32),
            pltpu.VMEM((2, PGC, IDXW), jnp.int32),
            pltpu.VMEM((NB, IDXW, W), jnp.float32),
            pltpu.VMEM_SHARED((NPAD, W), jnp.float32),
            tuple(pltpu.SemaphoreType.DMA for _ in range(2 * NB + 2)),
        ],
        compiler_params=pltpu.CompilerParams(use_tc_tiling_on_sc=False),
    )


def _sage_block(a0, a1, xa, wl, bl, wr):
    agg = a0[:, :D] + a1[:, :D]
    cnt = a0[:, D:D + 1] + a1[:, D:D + 1]
    x = xa[:, :D]
    mean = agg / jnp.maximum(cnt, 1.0)
    out = (jnp.dot(mean, wl, preferred_element_type=jnp.float32) + bl
           + jnp.dot(x, wr, preferred_element_type=jnp.float32))
    nrm = jnp.sqrt(jnp.sum(out * out, axis=1, keepdims=True))
    out = out / jnp.maximum(nrm, 1e-12)
    return jnp.maximum(out, 0.0)


def _layer_body(a0_ref, a1_ref, xa_ref, wl_ref, bl_ref, wr_ref, o_ref):
    out = _sage_block(a0_ref[...], a1_ref[...], xa_ref[...],
                      wl_ref[...], bl_ref[...], wr_ref[...])
    r = out.shape[0]
    o_ref[:, :D] = out
    col = lax.broadcasted_iota(jnp.int32, (r, W - D), 1)
    o_ref[:, D:W] = jnp.where(col == 0, 1.0, 0.0)


def _head_body(a0_ref, a1_ref, xa_ref, wl_ref, bl_ref, wr_ref,
               w0_ref, b0_ref, w1_ref, b1_ref, o_ref):
    x3 = _sage_block(a0_ref[...], a1_ref[...], xa_ref[...],
                     wl_ref[...], bl_ref[...], wr_ref[...])
    h = jnp.maximum(jnp.dot(x3, w0_ref[...],
                            preferred_element_type=jnp.float32)
                    + b0_ref[...], 0.0)
    o_ref[...] = (jnp.dot(h, w1_ref[...], preferred_element_type=jnp.float32)
                  + b1_ref[...])


_BR = 1280


def _row_spec(w, off=0):
    return pl.BlockSpec((_BR, w), lambda i, o=off: (i + o, 0))


def _full_spec(a, b):
    return pl.BlockSpec((a, b), lambda i: (0, 0))


def _make_tc_layer(interpret=False):
    return pl.pallas_call(
        _layer_body,
        grid=(NPAD // _BR,),
        in_specs=[
            _row_spec(W), _row_spec(W, NPAD // _BR), _row_spec(W),
            _full_spec(D, D), _full_spec(1, D), _full_spec(D, D),
        ],
        out_specs=_row_spec(W),
        out_shape=jax.ShapeDtypeStruct((NPAD, W), jnp.float32),
        interpret=interpret,
    )


def _make_tc_head(interpret=False):
    return pl.pallas_call(
        _head_body,
        grid=(NPAD // _BR,),
        in_specs=[
            _row_spec(W), _row_spec(W, NPAD // _BR), _row_spec(W),
            _full_spec(D, D), _full_spec(1, D), _full_spec(D, D),
            _full_spec(D, D), _full_spec(1, D),
            _full_spec(D, D), _full_spec(1, D),
        ],
        out_specs=_row_spec(D),
        out_shape=jax.ShapeDtypeStruct((NPAD, D), jnp.float32),
        interpret=interpret,
    )


def kernel(x, edge_index, Wl0, bl0, Wr0, Wl1, bl1, Wr1, Wl2, bl2, Wr2,
           Wlin0, blin0, Wlin1, blin1):
    xa = jnp.zeros((NPAD, W), jnp.float32)
    xa = xa.at[:N, :D].set(x)
    xa = xa.at[:N, D].set(1.0)
    src2d = edge_index[0].reshape(NW, CPW, IDXW)
    dst2d = edge_index[1].reshape(NW, CPW, IDXW)
    zeros_stage = jnp.zeros((RPT, W), jnp.float32)

    out_dim = Wlin1.shape[1]
    w1p = jnp.zeros((D, D), jnp.float32).at[:, :out_dim].set(Wlin1)
    b1p = jnp.zeros((1, D), jnp.float32).at[0, :out_dim].set(blin1)

    sc_agg = _make_sc_agg()
    tc_layer = _make_tc_layer()
    tc_head = _make_tc_head()

    layers = [(Wl0, bl0.reshape(1, D), Wr0),
              (Wl1, bl1.reshape(1, D), Wr1),
              (Wl2, bl2.reshape(1, D), Wr2)]

    for i, (wl, bl, wr) in enumerate(layers):
        partials = sc_agg(xa, src2d, dst2d, zeros_stage)
        if i < 2:
            xa = tc_layer(partials, partials, xa, wl, bl, wr)
        else:
            out = tc_head(partials, partials, xa, wl, bl, wr,
                          Wlin0, blin0.reshape(1, D), w1p, b1p)
    return out[:N, :out_dim]


# speedup vs baseline: 1.1552x; 1.1552x over previous
import functools

import jax
import jax.numpy as jnp
from jax import lax
from jax.experimental import pallas as pl
from jax.experimental.pallas import tpu as pltpu
from jax.experimental.pallas import tpu_sc as plsc

N = 10000
E = 320000
D = 128
W = 144
NPAD = 10240
NC = 2
NS = 16
NW = NC * NS
IDXW = 50
CPW = (E // IDXW) // NW
PGC = 40
NPG = CPW // PGC
NB = 4
RPT = NPAD // NS


def _sc_agg_body(xa_hbm, src_hbm, dst_hbm, zeros_hbm, out_hbm,
                 idx_s, idx_d, rows, shared, sem):
    cid = lax.axis_index("c")
    sid = lax.axis_index("s")
    wid = cid * NS + sid

    gsems = sem[:NB]
    isems = sem[NB:]

    pltpu.sync_copy(zeros_hbm, shared.at[pl.ds(sid * RPT, RPT)])
    pltpu.sync_copy(src_hbm.at[wid, pl.ds(0, PGC)], idx_s.at[0])
    pltpu.sync_copy(dst_hbm.at[wid, pl.ds(0, PGC)], idx_d.at[0])
    plsc.subcore_barrier()

    def gather(pb, k, buf):
        pltpu.async_copy(xa_hbm.at[idx_s.at[pb, k]], rows.at[buf],
                         gsems[buf])

    def wait_gather(pb, k, buf):
        pltpu.make_async_copy(xa_hbm.at[idx_s.at[pb, k]], rows.at[buf],
                              gsems[buf]).wait()

    for p in range(NPG):
        pb = p % 2
        if p > 0:
            pltpu.make_async_copy(src_hbm.at[wid, pl.ds(p * PGC, PGC)],
                                  idx_s.at[pb], isems[pb]).wait()
            pltpu.make_async_copy(dst_hbm.at[wid, pl.ds(p * PGC, PGC)],
                                  idx_d.at[pb], isems[pb]).wait()
        if p < NPG - 1:
            nb = (p + 1) % 2
            pltpu.async_copy(src_hbm.at[wid, pl.ds((p + 1) * PGC, PGC)],
                             idx_s.at[nb], isems[nb])
            pltpu.async_copy(dst_hbm.at[wid, pl.ds((p + 1) * PGC, PGC)],
                             idx_d.at[nb], isems[nb])

        for m in range(NB):
            gather(pb, m, m)

        def step(k, carry, pb=pb):
            for m in range(NB):
                c = NB * k + m
                wait_gather(pb, c, m)
                pltpu.sync_copy(rows.at[m], shared.at[idx_d.at[pb, c]],
                                add=True)
                gather(pb, c + NB, m)
            return carry

        lax.fori_loop(0, PGC // NB - 1, step, 0)
        for m in range(NB):
            c = PGC - NB + m
            wait_gather(pb, c, m)
            pltpu.sync_copy(rows.at[m], shared.at[idx_d.at[pb, c]],
                            add=True)
    plsc.subcore_barrier()

    pltpu.sync_copy(shared.at[pl.ds(sid * RPT, RPT)],
                    out_hbm.at[pl.ds(cid * NPAD + sid * RPT, RPT)])


def _make_sc_agg():
    mesh = plsc.VectorSubcoreMesh(core_axis_name="c", subcore_axis_name="s")
    return pl.kernel(
        _sc_agg_body,
        out_type=jax.ShapeDtypeStruct((2 * NPAD, W), jnp.float32),
        mesh=mesh,
        scratch_types=[
            pltpu.VMEM((2, PGC, IDXW), jnp.int32),
            pltpu.VMEM((2, PGC, IDXW), jnp.int32),
            pltpu.VMEM((NB, IDXW, W), jnp.float32),
            pltpu.VMEM_SHARED((NPAD, W), jnp.float32),
            tuple(pltpu.SemaphoreType.DMA for _ in range(NB + 2)),
        ],
        compiler_params=pltpu.CompilerParams(use_tc_tiling_on_sc=False),
    )


def _sage_block(a0, a1, xa, wl, bl, wr):
    agg = a0[:, :D] + a1[:, :D]
    cnt = a0[:, D:D + 1] + a1[:, D:D + 1]
    x = xa[:, :D]
    mean = agg / jnp.maximum(cnt, 1.0)
    out = (jnp.dot(mean, wl, preferred_element_type=jnp.float32) + bl
           + jnp.dot(x, wr, preferred_element_type=jnp.float32))
    nrm = jnp.sqrt(jnp.sum(out * out, axis=1, keepdims=True))
    out = out / jnp.maximum(nrm, 1e-12)
    return jnp.maximum(out, 0.0)


def _layer_body(a0_ref, a1_ref, xa_ref, wl_ref, bl_ref, wr_ref, o_ref):
    out = _sage_block(a0_ref[...], a1_ref[...], xa_ref[...],
                      wl_ref[...], bl_ref[...], wr_ref[...])
    r = out.shape[0]
    o_ref[:, :D] = out
    col = lax.broadcasted_iota(jnp.int32, (r, W - D), 1)
    o_ref[:, D:W] = jnp.where(col == 0, 1.0, 0.0)


def _head_body(a0_ref, a1_ref, xa_ref, wl_ref, bl_ref, wr_ref,
               w0_ref, b0_ref, w1_ref, b1_ref, o_ref):
    x3 = _sage_block(a0_ref[...], a1_ref[...], xa_ref[...],
                     wl_ref[...], bl_ref[...], wr_ref[...])
    h = jnp.maximum(jnp.dot(x3, w0_ref[...],
                            preferred_element_type=jnp.float32)
                    + b0_ref[...], 0.0)
    o_ref[...] = (jnp.dot(h, w1_ref[...], preferred_element_type=jnp.float32)
                  + b1_ref[...])


_BR = 1280


def _row_spec(w, off=0):
    return pl.BlockSpec((_BR, w), lambda i, o=off: (i + o, 0))


def _full_spec(a, b):
    return pl.BlockSpec((a, b), lambda i: (0, 0))


def _make_tc_layer(interpret=False):
    return pl.pallas_call(
        _layer_body,
        grid=(NPAD // _BR,),
        in_specs=[
            _row_spec(W), _row_spec(W, NPAD // _BR), _row_spec(W),
            _full_spec(D, D), _full_spec(1, D), _full_spec(D, D),
        ],
        out_specs=_row_spec(W),
        out_shape=jax.ShapeDtypeStruct((NPAD, W), jnp.float32),
        interpret=interpret,
    )


def _make_tc_head(interpret=False):
    return pl.pallas_call(
        _head_body,
        grid=(NPAD // _BR,),
        in_specs=[
            _row_spec(W), _row_spec(W, NPAD // _BR), _row_spec(W),
            _full_spec(D, D), _full_spec(1, D), _full_spec(D, D),
            _full_spec(D, D), _full_spec(1, D),
            _full_spec(D, D), _full_spec(1, D),
        ],
        out_specs=_row_spec(D),
        out_shape=jax.ShapeDtypeStruct((NPAD, D), jnp.float32),
        interpret=interpret,
    )


def kernel(x, edge_index, Wl0, bl0, Wr0, Wl1, bl1, Wr1, Wl2, bl2, Wr2,
           Wlin0, blin0, Wlin1, blin1):
    xa = jnp.zeros((NPAD, W), jnp.float32)
    xa = xa.at[:N, :D].set(x)
    xa = xa.at[:N, D].set(1.0)
    src2d = edge_index[0].reshape(NW, CPW, IDXW)
    dst2d = edge_index[1].reshape(NW, CPW, IDXW)
    zeros_stage = jnp.zeros((RPT, W), jnp.float32)

    out_dim = Wlin1.shape[1]
    w1p = jnp.zeros((D, D), jnp.float32).at[:, :out_dim].set(Wlin1)
    b1p = jnp.zeros((1, D), jnp.float32).at[0, :out_dim].set(blin1)

    sc_agg = _make_sc_agg()
    tc_layer = _make_tc_layer()
    tc_head = _make_tc_head()

    layers = [(Wl0, bl0.reshape(1, D), Wr0),
              (Wl1, bl1.reshape(1, D), Wr1),
              (Wl2, bl2.reshape(1, D), Wr2)]

    for i, (wl, bl, wr) in enumerate(layers):
        partials = sc_agg(xa, src2d, dst2d, zeros_stage)
        if i < 2:
            xa = tc_layer(partials, partials, xa, wl, bl, wr)
        else:
            out = tc_head(partials, partials, xa, wl, bl, wr,
                          Wlin0, blin0.reshape(1, D), w1p, b1p)
    return out[:N, :out_dim]


# speedup vs baseline: 1.1777x; 1.0195x over previous
import functools

import jax
import jax.numpy as jnp
from jax import lax
from jax.experimental import pallas as pl
from jax.experimental.pallas import tpu as pltpu
from jax.experimental.pallas import tpu_sc as plsc

N = 10000
E = 320000
D = 128
W = 144
NPAD = 10240
NC = 2
NS = 16
NW = NC * NS
IDXW = 50
CPW = (E // IDXW) // NW
PGC = 40
NPG = CPW // PGC
NB = 4
RPT = NPAD // NS


def _sc_agg_body(xa_hbm, edge_hbm, zeros_hbm, out_hbm,
                 idx_s, idx_d, rows, shared, sem):
    cid = lax.axis_index("c")
    sid = lax.axis_index("s")
    wid = cid * NS + sid

    gsems = sem[:NB]
    isems = sem[NB:]

    pltpu.sync_copy(zeros_hbm, shared.at[pl.ds(sid * RPT, RPT)])
    pltpu.sync_copy(edge_hbm.at[0, wid, pl.ds(0, PGC)], idx_s.at[0])
    pltpu.sync_copy(edge_hbm.at[1, wid, pl.ds(0, PGC)], idx_d.at[0])
    plsc.subcore_barrier()

    def gather(pb, k, buf):
        pltpu.async_copy(xa_hbm.at[idx_s.at[pb, k]], rows.at[buf],
                         gsems[buf])

    def wait_gather(pb, k, buf):
        pltpu.make_async_copy(xa_hbm.at[idx_s.at[pb, k]], rows.at[buf],
                              gsems[buf]).wait()

    for p in range(NPG):
        pb = p % 2
        if p > 0:
            pltpu.make_async_copy(edge_hbm.at[0, wid, pl.ds(p * PGC, PGC)],
                                  idx_s.at[pb], isems[pb]).wait()
            pltpu.make_async_copy(edge_hbm.at[1, wid, pl.ds(p * PGC, PGC)],
                                  idx_d.at[pb], isems[pb]).wait()
        if p < NPG - 1:
            nb = (p + 1) % 2
            pltpu.async_copy(edge_hbm.at[0, wid, pl.ds((p + 1) * PGC, PGC)],
                             idx_s.at[nb], isems[nb])
            pltpu.async_copy(edge_hbm.at[1, wid, pl.ds((p + 1) * PGC, PGC)],
                             idx_d.at[nb], isems[nb])

        for m in range(NB):
            gather(pb, m, m)

        def step(k, carry, pb=pb):
            for m in range(NB):
                c = NB * k + m
                wait_gather(pb, c, m)
                pltpu.sync_copy(rows.at[m], shared.at[idx_d.at[pb, c]],
                                add=True)
                gather(pb, c + NB, m)
            return carry

        lax.fori_loop(0, PGC // NB - 1, step, 0)
        for m in range(NB):
            c = PGC - NB + m
            wait_gather(pb, c, m)
            pltpu.sync_copy(rows.at[m], shared.at[idx_d.at[pb, c]],
                            add=True)
    plsc.subcore_barrier()

    pltpu.sync_copy(shared.at[pl.ds(sid * RPT, RPT)],
                    out_hbm.at[pl.ds(cid * NPAD + sid * RPT, RPT)])


def _make_sc_agg():
    mesh = plsc.VectorSubcoreMesh(core_axis_name="c", subcore_axis_name="s")
    return pl.kernel(
        _sc_agg_body,
        out_type=jax.ShapeDtypeStruct((2 * NPAD, W), jnp.float32),
        mesh=mesh,
        scratch_types=[
            pltpu.VMEM((2, PGC, IDXW), jnp.int32),
            pltpu.VMEM((2, PGC, IDXW), jnp.int32),
            pltpu.VMEM((NB, IDXW, W), jnp.float32),
            pltpu.VMEM_SHARED((NPAD, W), jnp.float32),
            tuple(pltpu.SemaphoreType.DMA for _ in range(NB + 2)),
        ],
        compiler_params=pltpu.CompilerParams(use_tc_tiling_on_sc=False),
    )


def _sage_block(a0, a1, xa, wl, bl, wr):
    agg = a0[:, :D] + a1[:, :D]
    cnt = a0[:, D:D + 1] + a1[:, D:D + 1]
    x = xa[:, :D]
    mean = agg / jnp.maximum(cnt, 1.0)
    out = (jnp.dot(mean, wl, preferred_element_type=jnp.float32) + bl
           + jnp.dot(x, wr, preferred_element_type=jnp.float32))
    nrm = jnp.sqrt(jnp.sum(out * out, axis=1, keepdims=True))
    out = out / jnp.maximum(nrm, 1e-12)
    return jnp.maximum(out, 0.0)


def _layer_body(a0_ref, a1_ref, xa_ref, wl_ref, bl_ref, wr_ref, o_ref):
    out = _sage_block(a0_ref[...], a1_ref[...], xa_ref[...],
                      wl_ref[...], bl_ref[...], wr_ref[...])
    r = out.shape[0]
    o_ref[:, :D] = out
    col = lax.broadcasted_iota(jnp.int32, (r, W - D), 1)
    o_ref[:, D:W] = jnp.where(col == 0, 1.0, 0.0)


def _head_body(a0_ref, a1_ref, xa_ref, wl_ref, bl_ref, wr_ref,
               w0_ref, b0_ref, w1_ref, b1_ref, o_ref):
    x3 = _sage_block(a0_ref[...], a1_ref[...], xa_ref[...],
                     wl_ref[...], bl_ref[...], wr_ref[...])
    h = jnp.maximum(jnp.dot(x3, w0_ref[...],
                            preferred_element_type=jnp.float32)
                    + b0_ref[...], 0.0)
    o_ref[...] = (jnp.dot(h, w1_ref[...], preferred_element_type=jnp.float32)
                  + b1_ref[...])


_BR = 1280


def _row_spec(w, off=0):
    return pl.BlockSpec((_BR, w), lambda i, o=off: (i + o, 0))


def _full_spec(a, b):
    return pl.BlockSpec((a, b), lambda i: (0, 0))


def _make_tc_layer(interpret=False):
    return pl.pallas_call(
        _layer_body,
        grid=(NPAD // _BR,),
        in_specs=[
            _row_spec(W), _row_spec(W, NPAD // _BR), _row_spec(W),
            _full_spec(D, D), _full_spec(1, D), _full_spec(D, D),
        ],
        out_specs=_row_spec(W),
        out_shape=jax.ShapeDtypeStruct((NPAD, W), jnp.float32),
        interpret=interpret,
    )


def _make_tc_head(interpret=False):
    return pl.pallas_call(
        _head_body,
        grid=(NPAD // _BR,),
        in_specs=[
            _row_spec(W), _row_spec(W, NPAD // _BR), _row_spec(W),
            _full_spec(D, D), _full_spec(1, D), _full_spec(D, D),
            _full_spec(D, D), _full_spec(1, D),
            _full_spec(D, D), _full_spec(1, D),
        ],
        out_specs=_row_spec(D),
        out_shape=jax.ShapeDtypeStruct((NPAD, D), jnp.float32),
        interpret=interpret,
    )


def kernel(x, edge_index, Wl0, bl0, Wr0, Wl1, bl1, Wr1, Wl2, bl2, Wr2,
           Wlin0, blin0, Wlin1, blin1):
    xa = jnp.zeros((NPAD, W), jnp.float32)
    xa = xa.at[:N, :D].set(x)
    xa = xa.at[:N, D].set(1.0)
    e4d = edge_index.reshape(2, NW, CPW, IDXW)
    zeros_stage = jnp.zeros((RPT, W), jnp.float32)

    out_dim = Wlin1.shape[1]
    w1p = jnp.zeros((D, D), jnp.float32).at[:, :out_dim].set(Wlin1)
    b1p = jnp.zeros((1, D), jnp.float32).at[0, :out_dim].set(blin1)

    sc_agg = _make_sc_agg()
    tc_layer = _make_tc_layer()
    tc_head = _make_tc_head()

    layers = [(Wl0, bl0.reshape(1, D), Wr0),
              (Wl1, bl1.reshape(1, D), Wr1),
              (Wl2, bl2.reshape(1, D), Wr2)]

    for i, (wl, bl, wr) in enumerate(layers):
        partials = sc_agg(xa, e4d, zeros_stage)
        if i < 2:
            xa = tc_layer(partials, partials, xa, wl, bl, wr)
        else:
            out = tc_head(partials, partials, xa, wl, bl, wr,
                          Wlin0, blin0.reshape(1, D), w1p, b1p)
    return out[:N, :out_dim]
